# baseline (speedup 1.0000x reference)
import functools

import jax
import jax.numpy as jnp
from jax.experimental import pallas as pl


def _square_distance(src, dst):
    dist = -2.0 * jnp.matmul(src, jnp.swapaxes(dst, 1, 2))
    dist = dist + jnp.sum(src ** 2, -1)[:, :, None]
    dist = dist + jnp.sum(dst ** 2, -1)[:, None, :]
    return dist


def _index_points(points, idx):
    return jax.vmap(lambda p, i: p[i])(points, idx)


def _fps(xyz, npoint):
    B, N, _ = xyz.shape
    def body(carry, _):
        distance, farthest = carry
        centroid = xyz[jnp.arange(B), farthest][:, None, :]
        d = jnp.sum((xyz - centroid) ** 2, -1)
        distance = jnp.minimum(distance, d)
        nf = jnp.argmax(distance, -1).astype(jnp.int32)
        return (distance, nf), farthest
    init = (jnp.full((B, N), 1e10, xyz.dtype), jnp.zeros((B,), jnp.int32))
    _, cents = jax.lax.scan(body, init, None, length=npoint)
    return jnp.swapaxes(cents, 0, 1)


def _query_ball_point(radius, nsample, xyz, new_xyz):
    N = xyz.shape[1]
    sq = _square_distance(new_xyz, xyz)
    gi = jnp.broadcast_to(jnp.arange(N), sq.shape)
    gi = jnp.where(sq > radius ** 2, N, gi)
    gi = jnp.sort(gi, axis=-1)[:, :, :nsample]
    first = gi[:, :, :1]
    gi = jnp.where(gi == N, first, gi)
    return gi


def _gelu(x):
    return jax.nn.gelu(x, approximate=False)


def _gn(x, gamma, beta, groups=4, eps=1e-5):
    sh = x.shape
    B, C = sh[0], sh[1]
    xg = x.reshape(B, groups, C // groups, -1)
    m = jnp.mean(xg, axis=(2, 3), keepdims=True)
    v = jnp.var(xg, axis=(2, 3), keepdims=True)
    xg = (xg - m) / jnp.sqrt(v + eps)
    x = xg.reshape(sh)
    bs = (1, C) + (1,) * (len(sh) - 2)
    return x * gamma.reshape(bs) + beta.reshape(bs)


def _conv1d(x, W, b):
    return jnp.einsum('oc,bcn->bon', W, x) + b[None, :, None]


def _invres(x, pos, p, radius, nsample):
    identity = x
    h = _gelu(_gn(_conv1d(x, p['w1'], p['b1']), p['g1'], p['be1']))
    pos_t = jnp.swapaxes(pos, 1, 2)
    idx = _query_ball_point(radius, nsample, pos_t, pos_t)
    gx = jnp.transpose(_index_points(jnp.swapaxes(h, 1, 2), idx), (0, 3, 1, 2))
    gx = _gn(gx * p['dw'][None, :, None, None], p['gdw'], p['bdw'])
    h = _gelu(jnp.max(gx, axis=-1))
    h = _gn(_conv1d(h, p['w2'], p['b2']), p['g2'], p['be2'])
    return h + identity


def _downsample_layer(x, pos, p, npoint, radius, nsample):
    pos_t = jnp.swapaxes(pos, 1, 2)
    if npoint < pos.shape[2]:
        fi = _fps(pos_t, npoint)
        new_pos = _index_points(pos_t, fi)
    else:
        new_pos = pos_t
    idx = _query_ball_point(radius, nsample, pos_t, new_pos)
    gp = _index_points(pos_t, idx)
    gpn = jnp.transpose(gp - new_pos[:, :, None, :], (0, 3, 1, 2))
    if x is not None:
        gx = jnp.transpose(_index_points(jnp.swapaxes(x, 1, 2), idx), (0, 3, 1, 2))
        feat = jnp.concatenate([gx, gpn], axis=1)
    else:
        feat = gpn
    B, C, S, K = feat.shape
    out = _gelu(_gn(_conv1d(feat.reshape(B, C, S * K), p['w'], p['b']), p['g'], p['be']))
    out = jnp.max(out.reshape(B, -1, S, K), axis=-1)
    return out, jnp.swapaxes(new_pos, 1, 2)


def _stage(x, pos, p, npoint, radius, nsample):
    x, pos = _downsample_layer(x, pos, p['down'], npoint, radius, nsample)
    x = _invres(x, pos, p['block'], radius, nsample)
    return x, pos


def _upsample_layer(x_up, pos_up, x_skip, pos_skip, p, radius, nsample):
    ps = jnp.swapaxes(pos_skip, 1, 2)
    pu = jnp.swapaxes(pos_up, 1, 2)
    sq = _square_distance(ps, pu)
    idx = jnp.argsort(sq, axis=-1)[:, :, :3]
    dist = jnp.take_along_axis(sq, idx, axis=-1)
    dr = 1.0 / (dist + 1e-8)
    w = dr / jnp.sum(dr, axis=2, keepdims=True)
    interp = jnp.sum(_index_points(jnp.swapaxes(x_up, 1, 2), idx) * w[..., None], axis=2)
    interp = jnp.swapaxes(interp, 1, 2)
    fused = jnp.concatenate([x_skip, interp], axis=1) if x_skip is not None else interp
    x = _gelu(_gn(_conv1d(fused, p['fw'], p['fb']), p['fg'], p['fbe']))
    return _invres(x, pos_skip, p['block'], radius, nsample)


def _head_kernel(x_ref, w_ref, b_ref, o_ref):
    o_ref[0] = (
        jnp.dot(w_ref[...], x_ref[0], preferred_element_type=jnp.float32)
        + b_ref[...]
    )


def _head_conv(x, W, b):
    B, C, N = x.shape
    O = W.shape[0]
    return pl.pallas_call(
        _head_kernel,
        grid=(B,),
        in_specs=[
            pl.BlockSpec((1, C, N), lambda b_: (b_, 0, 0)),
            pl.BlockSpec((O, C), lambda b_: (0, 0)),
            pl.BlockSpec((O, 1), lambda b_: (0, 0)),
        ],
        out_specs=pl.BlockSpec((1, O, N), lambda b_: (b_, 0, 0)),
        out_shape=jax.ShapeDtypeStruct((B, O, N), jnp.float32),
    )(x, W, b.reshape(O, 1))


def kernel(pos, params):
    x1, pos1 = _stage(None, pos, params['s1'], 512, 0.1, 32)
    x2, pos2 = _stage(x1, pos1, params['s2'], 128, 0.2, 32)
    x = _upsample_layer(x2, pos2, x1, pos1, params['u1'], 0.1, 32)
    x = _upsample_layer(x, pos1, None, pos, params['u0'], 0.1, 32)
    return _head_conv(x, params['head_w'], params['head_b'])


# speedup vs baseline: 6.0166x; 6.0166x over previous
import jax
import jax.numpy as jnp
from jax.experimental import pallas as pl
from jax.experimental.pallas import tpu as pltpu

_F32 = jnp.float32


def _bf(x):
    return x.astype(jnp.bfloat16)


def _gelu(x):
    return 0.5 * x * (1.0 + jax.lax.erf(x * 0.7071067811865476))


def _hilo(v):
    hi = _bf(v)
    r = v - hi.astype(_F32)
    mid = _bf(r)
    lo = _bf(r - mid.astype(_F32))
    return hi, mid, lo


def _gather_mm(m, parts):
    mb = _bf(m)
    hi, mid, lo = parts
    return ((jnp.dot(mb, hi, preferred_element_type=_F32)
             + jnp.dot(mb, mid, preferred_element_type=_F32))
            + jnp.dot(mb, lo, preferred_element_type=_F32))


def _cumsum_lanes(x):
    s, n = x.shape
    sh = 1
    while sh < n:
        x = x + jnp.concatenate(
            [jnp.zeros((s, sh), x.dtype), x[:, :n - sh]], axis=1)
        sh *= 2
    return x


def _s2_rows(p):
    q = p * p
    return (q[:, 0:1] + q[:, 1:2]) + q[:, 2:3]


def _s2_cols(p):
    q = p * p
    return (q[0:1, :] + q[1:2, :]) + q[2:3, :]


def _sq_dist(src_t, dst_cm, s2_src, s2_dst):
    mm = jnp.dot(_bf(src_t), _bf(dst_cm), preferred_element_type=_F32)
    return (-2.0 * mm + s2_src) + s2_dst


def _ball_rank(sq, r2):
    mask = (sq <= r2).astype(_F32)
    rank = _cumsum_lanes(mask)
    r = rank * mask
    cnt = rank[:, -1:]
    empty_oh = ((jax.lax.broadcasted_iota(jnp.int32, (1, sq.shape[1]), 1)
                 == sq.shape[1] - 1).astype(_F32) * (cnt == 0.0).astype(_F32))
    return r, cnt, empty_oh


def _gn_affine(ssum, ssq, count, gamma2d, beta2d, groups=4, eps=1e-5):
    c = ssum.shape[1]
    cg = c // groups
    n = float(count * cg)
    a_parts, b_parts = [], []
    for g in range(groups):
        sl = slice(g * cg, (g + 1) * cg)
        mean = jnp.sum(ssum[:, sl], axis=1, keepdims=True) / n
        var = jnp.sum(ssq[:, sl], axis=1, keepdims=True) / n - mean * mean
        inv = 1.0 / jnp.sqrt(var + eps)
        a = gamma2d[:, sl] * inv
        b = beta2d[:, sl] - mean * a
        a_parts.append(a)
        b_parts.append(b)
    return jnp.concatenate(a_parts, axis=1), jnp.concatenate(b_parts, axis=1)


def _gn_full(x, gamma2d, beta2d, groups=4):
    a, b = _gn_affine(jnp.sum(x, axis=0, keepdims=True),
                      jnp.sum(x * x, axis=0, keepdims=True),
                      x.shape[0], gamma2d, beta2d, groups)
    return x * a + b


def _invres_block(x, pos_t, pos_cm, p, r2, tq, nsample=32):
    s, _ = x.shape
    h_pre = jnp.dot(_bf(x), _bf(p['w1t']),
                    preferred_element_type=_F32) + p['b1']
    h = _gelu(_gn_full(h_pre, p['g1'], p['be1']))
    h2 = h * p['dw']
    mid = h2.shape[1]
    h2_parts = _hilo(h2)
    s2r = _s2_rows(pos_t)
    s2c = _s2_cols(pos_cm)
    mx_parts, mn_parts = [], []
    cntn = jnp.zeros((1, s), _F32)
    for t in range(s // tq):
        rows = slice(t * tq, (t + 1) * tq)
        sq = _sq_dist(pos_t[rows], pos_cm, s2r[rows], s2c)
        rk, cnt, empty_oh = _ball_rank(sq, r2)

        def kbody(k, carry, rk=rk, cnt=cnt, empty_oh=empty_oh):
            mx_t, mn_t, cn = carry
            kf = k.astype(_F32) + 1.0
            keff = jnp.where(kf <= cnt, kf, 1.0)
            m = (rk == keff).astype(_F32) + empty_oh
            gx = _gather_mm(m, h2_parts)
            return (jnp.maximum(mx_t, gx), jnp.minimum(mn_t, gx),
                    cn + jnp.sum(m, axis=0, keepdims=True))

        mx_t, mn_t, cntn = jax.lax.fori_loop(
            0, nsample, kbody,
            (jnp.full((tq, mid), -3.0e38, _F32),
             jnp.full((tq, mid), 3.0e38, _F32), cntn))
        mx_parts.append(mx_t)
        mn_parts.append(mn_t)
    mx = jnp.concatenate(mx_parts, axis=0) if len(mx_parts) > 1 else mx_parts[0]
    mn = jnp.concatenate(mn_parts, axis=0) if len(mn_parts) > 1 else mn_parts[0]
    wsum = jnp.sum(h2 * cntn.T, axis=0, keepdims=True)
    wssq = jnp.sum(h2 * h2 * cntn.T, axis=0, keepdims=True)
    a2, b2 = _gn_affine(wsum, wssq, s * nsample, p['gdw'], p['bdw'])
    pooled = jnp.where(a2 >= 0.0, mx * a2, mn * a2) + b2
    hp = _gelu(pooled)
    o2 = jnp.dot(_bf(hp), _bf(p['w2t']), preferred_element_type=_F32) + p['b2']
    return _gn_full(o2, p['g2'], p['be2']) + x


def _fps_kernel(pos_ref, npos_ref):
    bsz, _, n = pos_ref.shape
    npoint = npos_ref.shape[2]
    pos = pos_ref[...]
    p0, p1, p2 = pos[:, 0, :], pos[:, 1, :], pos[:, 2, :]
    iota = jax.lax.broadcasted_iota(jnp.int32, (1, n), 1).astype(_F32)
    iota_s = jax.lax.broadcasted_iota(jnp.int32, (1, npoint), 1)

    def body(i, carry):
        dist, far, cx, cy, cz = carry
        oh = (iota == far).astype(_F32)
        c0 = jnp.sum(p0 * oh, axis=1, keepdims=True)
        c1 = jnp.sum(p1 * oh, axis=1, keepdims=True)
        c2 = jnp.sum(p2 * oh, axis=1, keepdims=True)
        sel = iota_s == i
        cx = jnp.where(sel, c0, cx)
        cy = jnp.where(sel, c1, cy)
        cz = jnp.where(sel, c2, cz)
        t0, t1, t2 = p0 - c0, p1 - c1, p2 - c2
        d = (t0 * t0 + t1 * t1) + t2 * t2
        dist = jnp.minimum(dist, d)
        m = jnp.max(dist, axis=1, keepdims=True)
        far = jnp.min(jnp.where(dist == m, iota, jnp.float32(n)),
                      axis=1, keepdims=True)
        return dist, far, cx, cy, cz

    z = jnp.zeros((bsz, npoint), _F32)
    _, _, cx, cy, cz = jax.lax.fori_loop(
        0, npoint, body,
        (jnp.full((bsz, n), 1e10, _F32), jnp.zeros((bsz, 1), _F32), z, z, z))
    npos_ref[:, 0, :] = cx
    npos_ref[:, 1, :] = cy
    npos_ref[:, 2, :] = cz


def _fps(pos, npoint):
    bsz, _, n = pos.shape
    return pl.pallas_call(
        _fps_kernel,
        out_shape=jax.ShapeDtypeStruct((bsz, 3, npoint), _F32),
    )(pos)


def _downsample(pos_cm, npos_cm, x_pm, w_t, b2d, g2d, be2d, r2, oscr,
                nsample=32):
    n = pos_cm.shape[1]
    s = npos_cm.shape[1]
    cout = w_t.shape[1]
    npos_t = npos_cm.T
    pos_t = pos_cm.T
    p_parts = _hilo(pos_t)
    if x_pm is not None:
        x_parts = _hilo(x_pm)
    sq = _sq_dist(npos_t, pos_cm, _s2_rows(npos_t), _s2_cols(pos_cm))
    rk, cnt, empty_oh = _ball_rank(sq, r2)

    def kbody(k, carry):
        ssum, ssq = carry
        kf = k.astype(_F32) + 1.0
        keff = jnp.where(kf <= cnt, kf, 1.0)
        m = (rk == keff).astype(_F32) + empty_oh
        gp = _gather_mm(m, p_parts)
        gpn = gp - npos_t
        if x_pm is not None:
            gx = _gather_mm(m, x_parts)
            feat = jnp.concatenate([gx, gpn], axis=1)
        else:
            feat = gpn
        o = jnp.dot(_bf(feat), _bf(w_t), preferred_element_type=_F32) + b2d
        oscr[k] = o
        return (ssum + jnp.sum(o, axis=0, keepdims=True),
                ssq + jnp.sum(o * o, axis=0, keepdims=True))

    ssum, ssq = jax.lax.fori_loop(
        0, nsample, kbody,
        (jnp.zeros((1, cout), _F32), jnp.zeros((1, cout), _F32)))
    a, b = _gn_affine(ssum, ssq, s * nsample, g2d, be2d)
    ov = oscr[...]
    return jnp.max(_gelu(ov * a[None] + b[None]), axis=0)


def _stage1_kernel(pos_ref, npos_ref, pr, out_ref, oscr):
    pr = {k: v[...] for k, v in pr.items()}
    pos_cm = pos_ref[0]
    npos_cm = npos_ref[0]
    x = _downsample(pos_cm, npos_cm, None, pr['wt'], pr['b'], pr['g'],
                    pr['be'], 0.01, oscr)
    out_ref[0] = _invres_block(x, npos_cm.T, npos_cm, pr, 0.01, 512)


def _stage2_kernel(pos_ref, npos_ref, x_ref, pr, out_ref, oscr):
    pr = {k: v[...] for k, v in pr.items()}
    pos_cm = pos_ref[0]
    npos_cm = npos_ref[0]
    x = _downsample(pos_cm, npos_cm, x_ref[0], pr['wt'], pr['b'], pr['g'],
                    pr['be'], 0.04, oscr)
    out_ref[0] = _invres_block(x, npos_cm.T, npos_cm, pr, 0.04, 128)


def _top3_interp(sq, x_pm):
    nup = sq.shape[1]
    iota = jax.lax.broadcasted_iota(jnp.int32, (1, nup), 1).astype(_F32)
    big = jnp.float32(3.0e38)
    x_parts = _hilo(x_pm)
    cur = sq
    ms, gs = [], []
    for _ in range(3):
        m = jnp.min(cur, axis=1, keepdims=True)
        i = jnp.min(jnp.where(cur == m, iota, big), axis=1, keepdims=True)
        oh = (iota == i).astype(_F32)
        gs.append(_gather_mm(oh, x_parts))
        ms.append(m)
        cur = jnp.where(oh > 0.0, big, cur)
    dr1 = 1.0 / (ms[0] + 1e-8)
    dr2 = 1.0 / (ms[1] + 1e-8)
    dr3 = 1.0 / (ms[2] + 1e-8)
    sw = (dr1 + dr2) + dr3
    return ((dr1 / sw) * gs[0] + (dr2 / sw) * gs[1]) + (dr3 / sw) * gs[2]


def _u1_kernel(xup_ref, pup_ref, xskip_ref, pskip_ref, pr, out_ref):
    pr = {k: v[...] for k, v in pr.items()}
    ps_cm = pskip_ref[0]
    pu_cm = pup_ref[0]
    ps_t = ps_cm.T
    sq = _sq_dist(ps_t, pu_cm, _s2_rows(ps_t), _s2_cols(pu_cm))
    interp = _top3_interp(sq, xup_ref[0])
    fused = jnp.concatenate([xskip_ref[0], interp], axis=1)
    o = jnp.dot(_bf(fused), _bf(pr['fwt']),
                preferred_element_type=_F32) + pr['fb']
    x = _gelu(_gn_full(o, pr['fg'], pr['fbe']))
    out_ref[0] = _invres_block(x, ps_t, ps_cm, pr, 0.01, 512)


def _u0_kernel(xup_ref, pup_ref, pskip_ref, pr, out_ref):
    pr = {k: v[...] for k, v in pr.items()}
    ps_cm = pskip_ref[0]
    pu_cm = pup_ref[0]
    ps_t = ps_cm.T
    sq = _sq_dist(ps_t, pu_cm, _s2_rows(ps_t), _s2_cols(pu_cm))
    interp = _top3_interp(sq, xup_ref[0])
    o = jnp.dot(_bf(interp), _bf(pr['fwt']),
                preferred_element_type=_F32) + pr['fb']
    x = _gelu(_gn_full(o, pr['fg'], pr['fbe']))
    x = _invres_block(x, ps_t, ps_cm, pr, 0.01, 512)
    out_ref[0] = (jnp.dot(_bf(pr['hw']), _bf(x.T),
                          preferred_element_type=_F32) + pr['hb'])


def _r2(v):
    return v.reshape(1, -1)


def _prep_block(p):
    return {'w1t': p['w1'].T, 'b1': _r2(p['b1']), 'g1': _r2(p['g1']),
            'be1': _r2(p['be1']), 'dw': _r2(p['dw']), 'gdw': _r2(p['gdw']),
            'bdw': _r2(p['bdw']), 'w2t': p['w2'].T, 'b2': _r2(p['b2']),
            'g2': _r2(p['g2']), 'be2': _r2(p['be2'])}


def _batch_spec(shape):
    nd = len(shape)
    return pl.BlockSpec((1,) + shape[1:],
                        lambda b, nd=nd: (b,) + (0,) * (nd - 1))


def _rep_spec(x):
    nd = x.ndim
    return pl.BlockSpec(x.shape, lambda b, nd=nd: (0,) * nd)


def kernel(pos, params):
    bsz = pos.shape[0]
    npos1 = _fps(pos, 512)
    npos2 = _fps(npos1, 128)

    s1 = dict(_prep_block(params['s1']['block']))
    s1.update({'wt': params['s1']['down']['w'].T,
               'b': _r2(params['s1']['down']['b']),
               'g': _r2(params['s1']['down']['g']),
               'be': _r2(params['s1']['down']['be'])})
    x1 = pl.pallas_call(
        _stage1_kernel,
        grid=(bsz,),
        in_specs=[_batch_spec(pos.shape), _batch_spec(npos1.shape),
                  jax.tree.map(_rep_spec, s1)],
        out_specs=_batch_spec((bsz, 512, 32)),
        out_shape=jax.ShapeDtypeStruct((bsz, 512, 32), _F32),
        scratch_shapes=[pltpu.VMEM((32, 512, 32), _F32)],
    )(pos, npos1, s1)

    s2 = dict(_prep_block(params['s2']['block']))
    s2.update({'wt': params['s2']['down']['w'].T,
               'b': _r2(params['s2']['down']['b']),
               'g': _r2(params['s2']['down']['g']),
               'be': _r2(params['s2']['down']['be'])})
    x2 = pl.pallas_call(
        _stage2_kernel,
        grid=(bsz,),
        in_specs=[_batch_spec(npos1.shape), _batch_spec(npos2.shape),
                  _batch_spec(x1.shape), jax.tree.map(_rep_spec, s2)],
        out_specs=_batch_spec((bsz, 128, 64)),
        out_shape=jax.ShapeDtypeStruct((bsz, 128, 64), _F32),
        scratch_shapes=[pltpu.VMEM((32, 128, 64), _F32)],
    )(npos1, npos2, x1, s2)

    u1 = dict(_prep_block(params['u1']['block']))
    u1.update({'fwt': params['u1']['fw'].T, 'fb': _r2(params['u1']['fb']),
               'fg': _r2(params['u1']['fg']), 'fbe': _r2(params['u1']['fbe'])})
    xu1 = pl.pallas_call(
        _u1_kernel,
        grid=(bsz,),
        in_specs=[_batch_spec(x2.shape), _batch_spec(npos2.shape),
                  _batch_spec(x1.shape), _batch_spec(npos1.shape),
                  jax.tree.map(_rep_spec, u1)],
        out_specs=_batch_spec((bsz, 512, 32)),
        out_shape=jax.ShapeDtypeStruct((bsz, 512, 32), _F32),
    )(x2, npos2, x1, npos1, u1)

    u0 = dict(_prep_block(params['u0']['block']))
    u0.update({'fwt': params['u0']['fw'].T, 'fb': _r2(params['u0']['fb']),
               'fg': _r2(params['u0']['fg']), 'fbe': _r2(params['u0']['fbe']),
               'hw': params['head_w'], 'hb': params['head_b'].reshape(-1, 1)})
    out = pl.pallas_call(
        _u0_kernel,
        grid=(bsz,),
        in_specs=[_batch_spec(xu1.shape), _batch_spec(npos1.shape),
                  _batch_spec(pos.shape), jax.tree.map(_rep_spec, u0)],
        out_specs=_batch_spec((bsz, 13, 2048)),
        out_shape=jax.ShapeDtypeStruct((bsz, 13, 2048), _F32),
    )(xu1, npos1, pos, u0)
    return out
